# baseline (speedup 1.0000x reference)
import functools

import jax
import jax.numpy as jnp
from jax.experimental import pallas as pl
from jax.experimental.pallas import tpu as pltpu

_NUM_CLASSES = 80
_IOU_THR = 0.5
_SCORE_THR = 0.25
_MAX_PER_CLASS = 100
_MAX_TOTAL = 100
_K_CAND = 200
_LANES = 128

_PCALL = functools.partial(pl.pallas_call)


def _nms_stage1_kernel(class_ref, conf_ref, boxes_ref, s_out_ref, b_out_ref):
    cls_p = class_ref[0, 0]
    conf = conf_ref[0]
    raw = cls_p * conf
    s0 = jnp.where(raw > _SCORE_THR, raw, -1.0)

    def bis_body(_, st):
        lo, hi = st
        mid = 0.5 * (lo + hi)
        cnt = jnp.sum((s0 >= mid).astype(jnp.int32))
        ge = cnt >= _K_CAND
        lo2 = jnp.where(ge, mid, lo)
        hi2 = jnp.where(ge, hi, mid)
        return (lo2, hi2)

    lo, _ = jax.lax.fori_loop(
        0, 50, bis_body, (jnp.float32(_SCORE_THR), jnp.float32(1.5)))
    s_nms = jnp.where(s0 >= lo, s0, -1.0)

    y1 = boxes_ref[0, 0]
    x1 = boxes_ref[0, 1]
    y2 = boxes_ref[0, 2]
    x2 = boxes_ref[0, 3]
    area = jnp.maximum(y2 - y1, 0.0) * jnp.maximum(x2 - x1, 0.0)
    lane = jax.lax.broadcasted_iota(jnp.int32, (1, _LANES), 1)

    def body(i, st):
        s, os_, ob0, ob1, ob2, ob3 = st
        m = jnp.max(s)
        valid = m > 0.0
        oh = (s == m).astype(jnp.float32)
        sy1 = jnp.sum(oh * y1)
        sx1 = jnp.sum(oh * x1)
        sy2 = jnp.sum(oh * y2)
        sx2 = jnp.sum(oh * x2)
        sarea = jnp.maximum(sy2 - sy1, 0.0) * jnp.maximum(sx2 - sx1, 0.0)
        iy1 = jnp.maximum(y1, sy1)
        ix1 = jnp.maximum(x1, sx1)
        iy2 = jnp.minimum(y2, sy2)
        ix2 = jnp.minimum(x2, sx2)
        inter = jnp.maximum(iy2 - iy1, 0.0) * jnp.maximum(ix2 - ix1, 0.0)
        union = area + sarea - inter
        iou = inter / jnp.maximum(union, 1e-9)
        s = jnp.where((iou > _IOU_THR) | (oh > 0.0), -1.0, s)
        pick = (lane == i).astype(jnp.float32)
        os_ = os_ + jnp.where(valid, m, 0.0) * pick
        ob0 = ob0 + jnp.where(valid, sy1, 0.0) * pick
        ob1 = ob1 + jnp.where(valid, sx1, 0.0) * pick
        ob2 = ob2 + jnp.where(valid, sy2, 0.0) * pick
        ob3 = ob3 + jnp.where(valid, sx2, 0.0) * pick
        return (s, os_, ob0, ob1, ob2, ob3)

    z = jnp.zeros((1, _LANES), jnp.float32)
    _, os_, ob0, ob1, ob2, ob3 = jax.lax.fori_loop(
        0, _MAX_PER_CLASS, body, (s_nms, z, z, z, z, z))
    s_out_ref[...] = os_.reshape(1, 1, 1, _LANES)
    b_out_ref[...] = jnp.concatenate(
        [ob0, ob1, ob2, ob3], axis=0).reshape(1, 1, 4, _LANES)


def _merge_stage2_kernel(s_ref, b_ref, scr_ref, cls_ref, box_ref, val_ref):
    S = s_ref[0]
    row = jax.lax.broadcasted_iota(
        jnp.int32, (_NUM_CLASSES, _LANES), 0).astype(jnp.float32)
    lane = jax.lax.broadcasted_iota(jnp.int32, (1, _LANES), 1)
    n_valid = jnp.minimum(jnp.sum((S > 0.0).astype(jnp.int32)), _MAX_TOTAL)

    def body(i, st):
        s, oscr, ocls, ob0, ob1, ob2, ob3 = st
        m = jnp.max(s)
        valid = m > 0.0
        oh = (s == m).astype(jnp.float32)
        c = jnp.sum(oh * row)
        sy1 = jnp.sum(oh * b_ref[0, :, 0, :])
        sx1 = jnp.sum(oh * b_ref[0, :, 1, :])
        sy2 = jnp.sum(oh * b_ref[0, :, 2, :])
        sx2 = jnp.sum(oh * b_ref[0, :, 3, :])
        s = jnp.where(oh > 0.0, -1.0, s)
        pick = (lane == i).astype(jnp.float32)
        oscr = oscr + jnp.where(valid, m, 0.0) * pick
        ocls = ocls + jnp.where(valid, c, 0.0) * pick
        ob0 = ob0 + jnp.where(valid, sy1, 0.0) * pick
        ob1 = ob1 + jnp.where(valid, sx1, 0.0) * pick
        ob2 = ob2 + jnp.where(valid, sy2, 0.0) * pick
        ob3 = ob3 + jnp.where(valid, sx2, 0.0) * pick
        return (s, oscr, ocls, ob0, ob1, ob2, ob3)

    z = jnp.zeros((1, _LANES), jnp.float32)
    _, oscr, ocls, ob0, ob1, ob2, ob3 = jax.lax.fori_loop(
        0, _MAX_TOTAL, body, (S, z, z, z, z, z, z))
    scr_ref[...] = oscr.reshape(1, 1, _LANES)
    cls_ref[...] = ocls.reshape(1, 1, _LANES)
    box_ref[...] = jnp.concatenate(
        [ob0, ob1, ob2, ob3], axis=0).reshape(1, 4, _LANES)
    val_ref[...] = jnp.full((1, 1, 1), n_valid, jnp.int32)


def kernel(p0_boxes, p0_conf, p0_class, p1_boxes, p1_conf, p1_class,
           p2_boxes, p2_conf, p2_class):
    B = p0_boxes.shape[0]
    C = _NUM_CLASSES
    bboxes = jnp.concatenate(
        [p.reshape(B, -1, 4) for p in (p0_boxes, p1_boxes, p2_boxes)], axis=1)
    box_conf = jnp.concatenate(
        [p.reshape(B, -1) for p in (p0_conf, p1_conf, p2_conf)], axis=1)
    box_class = jnp.concatenate(
        [p.reshape(B, -1, C) for p in (p0_class, p1_class, p2_class)], axis=1)
    N = bboxes.shape[1]
    pad = (-N) % _LANES
    if pad:
        bboxes = jnp.pad(bboxes, ((0, 0), (0, pad), (0, 0)))
        box_conf = jnp.pad(box_conf, ((0, 0), (0, pad)))
        box_class = jnp.pad(box_class, ((0, 0), (0, pad), (0, 0)))
    Np = N + pad
    rows = Np // _LANES

    class_t = box_class.transpose(0, 2, 1).reshape(B, C, rows, _LANES)
    conf_t = box_conf.reshape(B, rows, _LANES)
    boxes_t = bboxes.transpose(0, 2, 1).reshape(B, 4, rows, _LANES)

    s_all, b_all = _PCALL(
        _nms_stage1_kernel,
        grid=(B, C),
        in_specs=[
            pl.BlockSpec((1, 1, rows, _LANES), lambda b, c: (b, c, 0, 0)),
            pl.BlockSpec((1, rows, _LANES), lambda b, c: (b, 0, 0)),
            pl.BlockSpec((1, 4, rows, _LANES), lambda b, c: (b, 0, 0, 0)),
        ],
        out_specs=[
            pl.BlockSpec((1, 1, 1, _LANES), lambda b, c: (b, c, 0, 0)),
            pl.BlockSpec((1, 1, 4, _LANES), lambda b, c: (b, c, 0, 0)),
        ],
        out_shape=[
            jax.ShapeDtypeStruct((B, C, 1, _LANES), jnp.float32),
            jax.ShapeDtypeStruct((B, C, 4, _LANES), jnp.float32),
        ],
        compiler_params=pltpu.CompilerParams(
            dimension_semantics=("parallel", "parallel")),
    )(class_t, conf_t, boxes_t)
    s_all = s_all.reshape(B, C, _LANES)

    scr, cls, box, val = _PCALL(
        _merge_stage2_kernel,
        grid=(B,),
        in_specs=[
            pl.BlockSpec((1, C, _LANES), lambda b: (b, 0, 0)),
            pl.BlockSpec((1, C, 4, _LANES), lambda b: (b, 0, 0, 0)),
        ],
        out_specs=[
            pl.BlockSpec((1, 1, _LANES), lambda b: (b, 0, 0)),
            pl.BlockSpec((1, 1, _LANES), lambda b: (b, 0, 0)),
            pl.BlockSpec((1, 4, _LANES), lambda b: (b, 0, 0)),
            pl.BlockSpec((1, 1, 1), lambda b: (b, 0, 0)),
        ],
        out_shape=[
            jax.ShapeDtypeStruct((B, 1, _LANES), jnp.float32),
            jax.ShapeDtypeStruct((B, 1, _LANES), jnp.float32),
            jax.ShapeDtypeStruct((B, 4, _LANES), jnp.float32),
            jax.ShapeDtypeStruct((B, 1, 1), jnp.int32),
        ],
        compiler_params=pltpu.CompilerParams(
            dimension_semantics=("parallel",)),
    )(s_all, b_all)

    boxes_o = box[:, :, :_MAX_TOTAL].transpose(0, 2, 1)
    scores_o = scr[:, 0, :_MAX_TOTAL]
    classes_o = cls[:, 0, :_MAX_TOTAL]
    valid_o = val[:, 0, 0]
    return boxes_o, scores_o, classes_o, valid_o


# speedup vs baseline: 1.4912x; 1.4912x over previous
import functools

import jax
import jax.numpy as jnp
from jax.experimental import pallas as pl
from jax.experimental.pallas import tpu as pltpu

_NUM_CLASSES = 80
_IOU_THR = 0.5
_SCORE_THR = 0.25
_MAX_PER_CLASS = 100
_MAX_TOTAL = 100
_K_CAND = 200
_LANES = 128

_PCALL = functools.partial(pl.pallas_call)


_CGRP = 4


def _nms_stage1_kernel(class_ref, conf_ref, boxes_ref, s_out_ref, b_out_ref):
    conf = conf_ref[0]
    s0s = tuple(
        jnp.where(class_ref[0, g] * conf > _SCORE_THR,
                  class_ref[0, g] * conf, -1.0)
        for g in range(_CGRP))

    def bis_body(_, st):
        los, his = st
        nlo, nhi = [], []
        for g in range(_CGRP):
            mid = 0.5 * (los[g] + his[g])
            cnt = jnp.sum((s0s[g] >= mid).astype(jnp.int32))
            ge = cnt >= _K_CAND
            nlo.append(jnp.where(ge, mid, los[g]))
            nhi.append(jnp.where(ge, his[g], mid))
        return (tuple(nlo), tuple(nhi))

    lo0 = (jnp.float32(_SCORE_THR),) * _CGRP
    hi0 = (jnp.float32(1.5),) * _CGRP
    los, _ = jax.lax.fori_loop(0, 50, bis_body, (lo0, hi0))
    s_init = tuple(
        jnp.where(s0s[g] >= los[g], s0s[g], -1.0) for g in range(_CGRP))

    y1 = boxes_ref[0, 0]
    x1 = boxes_ref[0, 1]
    y2 = boxes_ref[0, 2]
    x2 = boxes_ref[0, 3]
    area = jnp.maximum(y2 - y1, 0.0) * jnp.maximum(x2 - x1, 0.0)
    lane = jax.lax.broadcasted_iota(jnp.int32, (1, _LANES), 1)

    def body(i, st):
        ss, oss, obs = st
        pick = (lane == i).astype(jnp.float32)
        nss, noss, nobs = [], [], []
        for g in range(_CGRP):
            s = ss[g]
            m = jnp.max(s)
            valid = m > 0.0
            oh = (s == m).astype(jnp.float32)
            sy1 = jnp.sum(oh * y1)
            sx1 = jnp.sum(oh * x1)
            sy2 = jnp.sum(oh * y2)
            sx2 = jnp.sum(oh * x2)
            sarea = jnp.maximum(sy2 - sy1, 0.0) * jnp.maximum(sx2 - sx1, 0.0)
            iy1 = jnp.maximum(y1, sy1)
            ix1 = jnp.maximum(x1, sx1)
            iy2 = jnp.minimum(y2, sy2)
            ix2 = jnp.minimum(x2, sx2)
            inter = jnp.maximum(iy2 - iy1, 0.0) * jnp.maximum(ix2 - ix1, 0.0)
            union = area + sarea - inter
            iou = inter / jnp.maximum(union, 1e-9)
            nss.append(jnp.where((iou > _IOU_THR) | (oh > 0.0), -1.0, s))
            os_, (ob0, ob1, ob2, ob3) = oss[g], obs[g]
            noss.append(os_ + jnp.where(valid, m, 0.0) * pick)
            nobs.append((ob0 + jnp.where(valid, sy1, 0.0) * pick,
                         ob1 + jnp.where(valid, sx1, 0.0) * pick,
                         ob2 + jnp.where(valid, sy2, 0.0) * pick,
                         ob3 + jnp.where(valid, sx2, 0.0) * pick))
        return (tuple(nss), tuple(noss), tuple(nobs))

    z = jnp.zeros((1, _LANES), jnp.float32)
    st0 = (s_init, (z,) * _CGRP, ((z, z, z, z),) * _CGRP)
    _, oss, obs = jax.lax.fori_loop(0, _MAX_PER_CLASS, body, st0)
    s_out_ref[...] = jnp.concatenate(oss, axis=0).reshape(1, 1, _CGRP, _LANES)
    b_out_ref[...] = jnp.concatenate(
        [jnp.concatenate(obs[g], axis=0) for g in range(_CGRP)],
        axis=0).reshape(1, 1, _CGRP * 4, _LANES)


def _merge_stage2_kernel(s_ref, b_ref, scr_ref, cls_ref, box_ref, val_ref):
    S = s_ref[0]
    row = jax.lax.broadcasted_iota(
        jnp.int32, (_NUM_CLASSES, _LANES), 0).astype(jnp.float32)
    lane = jax.lax.broadcasted_iota(jnp.int32, (1, _LANES), 1)
    n_valid = jnp.minimum(jnp.sum((S > 0.0).astype(jnp.int32)), _MAX_TOTAL)

    def body(i, st):
        s, oscr, ocls, ob0, ob1, ob2, ob3 = st
        m = jnp.max(s)
        valid = m > 0.0
        oh = (s == m).astype(jnp.float32)
        c = jnp.sum(oh * row)
        sy1 = jnp.sum(oh * b_ref[0, :, 0, :])
        sx1 = jnp.sum(oh * b_ref[0, :, 1, :])
        sy2 = jnp.sum(oh * b_ref[0, :, 2, :])
        sx2 = jnp.sum(oh * b_ref[0, :, 3, :])
        s = jnp.where(oh > 0.0, -1.0, s)
        pick = (lane == i).astype(jnp.float32)
        oscr = oscr + jnp.where(valid, m, 0.0) * pick
        ocls = ocls + jnp.where(valid, c, 0.0) * pick
        ob0 = ob0 + jnp.where(valid, sy1, 0.0) * pick
        ob1 = ob1 + jnp.where(valid, sx1, 0.0) * pick
        ob2 = ob2 + jnp.where(valid, sy2, 0.0) * pick
        ob3 = ob3 + jnp.where(valid, sx2, 0.0) * pick
        return (s, oscr, ocls, ob0, ob1, ob2, ob3)

    z = jnp.zeros((1, _LANES), jnp.float32)
    _, oscr, ocls, ob0, ob1, ob2, ob3 = jax.lax.fori_loop(
        0, _MAX_TOTAL, body, (S, z, z, z, z, z, z))
    scr_ref[...] = oscr.reshape(1, 1, _LANES)
    cls_ref[...] = ocls.reshape(1, 1, _LANES)
    box_ref[...] = jnp.concatenate(
        [ob0, ob1, ob2, ob3], axis=0).reshape(1, 4, _LANES)
    val_ref[...] = jnp.full((1, 1, 1), n_valid, jnp.int32)


def kernel(p0_boxes, p0_conf, p0_class, p1_boxes, p1_conf, p1_class,
           p2_boxes, p2_conf, p2_class):
    B = p0_boxes.shape[0]
    C = _NUM_CLASSES
    bboxes = jnp.concatenate(
        [p.reshape(B, -1, 4) for p in (p0_boxes, p1_boxes, p2_boxes)], axis=1)
    box_conf = jnp.concatenate(
        [p.reshape(B, -1) for p in (p0_conf, p1_conf, p2_conf)], axis=1)
    box_class = jnp.concatenate(
        [p.reshape(B, -1, C) for p in (p0_class, p1_class, p2_class)], axis=1)
    N = bboxes.shape[1]
    pad = (-N) % _LANES
    if pad:
        bboxes = jnp.pad(bboxes, ((0, 0), (0, pad), (0, 0)))
        box_conf = jnp.pad(box_conf, ((0, 0), (0, pad)))
        box_class = jnp.pad(box_class, ((0, 0), (0, pad), (0, 0)))
    Np = N + pad
    rows = Np // _LANES

    class_t = box_class.transpose(0, 2, 1).reshape(B, C, rows, _LANES)
    conf_t = box_conf.reshape(B, rows, _LANES)
    boxes_t = bboxes.transpose(0, 2, 1).reshape(B, 4, rows, _LANES)

    s_all, b_all = _PCALL(
        _nms_stage1_kernel,
        grid=(B, C // _CGRP),
        in_specs=[
            pl.BlockSpec((1, _CGRP, rows, _LANES), lambda b, c: (b, c, 0, 0)),
            pl.BlockSpec((1, rows, _LANES), lambda b, c: (b, 0, 0)),
            pl.BlockSpec((1, 4, rows, _LANES), lambda b, c: (b, 0, 0, 0)),
        ],
        out_specs=[
            pl.BlockSpec((1, 1, _CGRP, _LANES), lambda b, c: (b, c, 0, 0)),
            pl.BlockSpec((1, 1, _CGRP * 4, _LANES), lambda b, c: (b, c, 0, 0)),
        ],
        out_shape=[
            jax.ShapeDtypeStruct((B, C // _CGRP, _CGRP, _LANES), jnp.float32),
            jax.ShapeDtypeStruct(
                (B, C // _CGRP, _CGRP * 4, _LANES), jnp.float32),
        ],
        compiler_params=pltpu.CompilerParams(
            dimension_semantics=("parallel", "parallel")),
    )(class_t, conf_t, boxes_t)
    s_all = s_all.reshape(B, C, _LANES)
    b_all = b_all.reshape(B, C, 4, _LANES)

    scr, cls, box, val = _PCALL(
        _merge_stage2_kernel,
        grid=(B,),
        in_specs=[
            pl.BlockSpec((1, C, _LANES), lambda b: (b, 0, 0)),
            pl.BlockSpec((1, C, 4, _LANES), lambda b: (b, 0, 0, 0)),
        ],
        out_specs=[
            pl.BlockSpec((1, 1, _LANES), lambda b: (b, 0, 0)),
            pl.BlockSpec((1, 1, _LANES), lambda b: (b, 0, 0)),
            pl.BlockSpec((1, 4, _LANES), lambda b: (b, 0, 0)),
            pl.BlockSpec((1, 1, 1), lambda b: (b, 0, 0)),
        ],
        out_shape=[
            jax.ShapeDtypeStruct((B, 1, _LANES), jnp.float32),
            jax.ShapeDtypeStruct((B, 1, _LANES), jnp.float32),
            jax.ShapeDtypeStruct((B, 4, _LANES), jnp.float32),
            jax.ShapeDtypeStruct((B, 1, 1), jnp.int32),
        ],
        compiler_params=pltpu.CompilerParams(
            dimension_semantics=("parallel",)),
    )(s_all, b_all)

    boxes_o = box[:, :, :_MAX_TOTAL].transpose(0, 2, 1)
    scores_o = scr[:, 0, :_MAX_TOTAL]
    classes_o = cls[:, 0, :_MAX_TOTAL]
    valid_o = val[:, 0, 0]
    return boxes_o, scores_o, classes_o, valid_o


# speedup vs baseline: 1.6276x; 1.0915x over previous
import functools

import jax
import jax.numpy as jnp
from jax.experimental import pallas as pl
from jax.experimental.pallas import tpu as pltpu

_NUM_CLASSES = 80
_IOU_THR = 0.5
_SCORE_THR = 0.25
_MAX_PER_CLASS = 100
_MAX_TOTAL = 100
_K_CAND = 200
_LANES = 128

_PCALL = functools.partial(pl.pallas_call)


_CGRP = 8


def _nms_stage1_kernel(class_ref, conf_ref, boxes_ref, s_out_ref, b_out_ref):
    conf = conf_ref[0]
    s0s = tuple(
        jnp.where(class_ref[0, g] * conf > _SCORE_THR,
                  class_ref[0, g] * conf, -1.0)
        for g in range(_CGRP))

    def bis_body(_, st):
        los, his = st
        nlo, nhi = [], []
        for g in range(_CGRP):
            mid = 0.5 * (los[g] + his[g])
            cnt = jnp.sum((s0s[g] >= mid).astype(jnp.int32))
            ge = cnt >= _K_CAND
            nlo.append(jnp.where(ge, mid, los[g]))
            nhi.append(jnp.where(ge, his[g], mid))
        return (tuple(nlo), tuple(nhi))

    lo0 = (jnp.float32(_SCORE_THR),) * _CGRP
    hi0 = (jnp.float32(1.5),) * _CGRP
    los, _ = jax.lax.fori_loop(0, 50, bis_body, (lo0, hi0))
    s_init = tuple(
        jnp.where(s0s[g] >= los[g], s0s[g], -1.0) for g in range(_CGRP))

    y1 = boxes_ref[0, 0]
    x1 = boxes_ref[0, 1]
    y2 = boxes_ref[0, 2]
    x2 = boxes_ref[0, 3]
    area = jnp.maximum(y2 - y1, 0.0) * jnp.maximum(x2 - x1, 0.0)
    lane = jax.lax.broadcasted_iota(jnp.int32, (1, _LANES), 1)

    def body(i, st):
        ss, oss, obs = st
        pick = (lane == i).astype(jnp.float32)
        nss, noss, nobs = [], [], []
        for g in range(_CGRP):
            s = ss[g]
            m = jnp.max(s)
            valid = m > 0.0
            oh = (s == m).astype(jnp.float32)
            sy1 = jnp.sum(oh * y1)
            sx1 = jnp.sum(oh * x1)
            sy2 = jnp.sum(oh * y2)
            sx2 = jnp.sum(oh * x2)
            sarea = jnp.maximum(sy2 - sy1, 0.0) * jnp.maximum(sx2 - sx1, 0.0)
            iy1 = jnp.maximum(y1, sy1)
            ix1 = jnp.maximum(x1, sx1)
            iy2 = jnp.minimum(y2, sy2)
            ix2 = jnp.minimum(x2, sx2)
            inter = jnp.maximum(iy2 - iy1, 0.0) * jnp.maximum(ix2 - ix1, 0.0)
            union = area + sarea - inter
            iou = inter / jnp.maximum(union, 1e-9)
            nss.append(jnp.where((iou > _IOU_THR) | (oh > 0.0), -1.0, s))
            os_, (ob0, ob1, ob2, ob3) = oss[g], obs[g]
            noss.append(os_ + jnp.where(valid, m, 0.0) * pick)
            nobs.append((ob0 + jnp.where(valid, sy1, 0.0) * pick,
                         ob1 + jnp.where(valid, sx1, 0.0) * pick,
                         ob2 + jnp.where(valid, sy2, 0.0) * pick,
                         ob3 + jnp.where(valid, sx2, 0.0) * pick))
        return (tuple(nss), tuple(noss), tuple(nobs))

    z = jnp.zeros((1, _LANES), jnp.float32)
    st0 = (s_init, (z,) * _CGRP, ((z, z, z, z),) * _CGRP)
    _, oss, obs = jax.lax.fori_loop(0, _MAX_PER_CLASS, body, st0)
    s_out_ref[...] = jnp.concatenate(oss, axis=0).reshape(1, 1, _CGRP, _LANES)
    b_out_ref[...] = jnp.concatenate(
        [jnp.concatenate(obs[g], axis=0) for g in range(_CGRP)],
        axis=0).reshape(1, 1, _CGRP * 4, _LANES)


def _merge_stage2_kernel(s_ref, b_ref, scr_ref, cls_ref, box_ref, val_ref):
    S = s_ref[0]
    row = jax.lax.broadcasted_iota(
        jnp.int32, (_NUM_CLASSES, _LANES), 0).astype(jnp.float32)
    lane = jax.lax.broadcasted_iota(jnp.int32, (1, _LANES), 1)
    n_valid = jnp.minimum(jnp.sum((S > 0.0).astype(jnp.int32)), _MAX_TOTAL)

    def body(i, st):
        s, oscr, ocls, ob0, ob1, ob2, ob3 = st
        m = jnp.max(s)
        valid = m > 0.0
        oh = (s == m).astype(jnp.float32)
        c = jnp.sum(oh * row)
        sy1 = jnp.sum(oh * b_ref[0, :, 0, :])
        sx1 = jnp.sum(oh * b_ref[0, :, 1, :])
        sy2 = jnp.sum(oh * b_ref[0, :, 2, :])
        sx2 = jnp.sum(oh * b_ref[0, :, 3, :])
        s = jnp.where(oh > 0.0, -1.0, s)
        pick = (lane == i).astype(jnp.float32)
        oscr = oscr + jnp.where(valid, m, 0.0) * pick
        ocls = ocls + jnp.where(valid, c, 0.0) * pick
        ob0 = ob0 + jnp.where(valid, sy1, 0.0) * pick
        ob1 = ob1 + jnp.where(valid, sx1, 0.0) * pick
        ob2 = ob2 + jnp.where(valid, sy2, 0.0) * pick
        ob3 = ob3 + jnp.where(valid, sx2, 0.0) * pick
        return (s, oscr, ocls, ob0, ob1, ob2, ob3)

    z = jnp.zeros((1, _LANES), jnp.float32)
    _, oscr, ocls, ob0, ob1, ob2, ob3 = jax.lax.fori_loop(
        0, _MAX_TOTAL, body, (S, z, z, z, z, z, z))
    scr_ref[...] = oscr.reshape(1, 1, _LANES)
    cls_ref[...] = ocls.reshape(1, 1, _LANES)
    box_ref[...] = jnp.concatenate(
        [ob0, ob1, ob2, ob3], axis=0).reshape(1, 4, _LANES)
    val_ref[...] = jnp.full((1, 1, 1), n_valid, jnp.int32)


def kernel(p0_boxes, p0_conf, p0_class, p1_boxes, p1_conf, p1_class,
           p2_boxes, p2_conf, p2_class):
    B = p0_boxes.shape[0]
    C = _NUM_CLASSES
    bboxes = jnp.concatenate(
        [p.reshape(B, -1, 4) for p in (p0_boxes, p1_boxes, p2_boxes)], axis=1)
    box_conf = jnp.concatenate(
        [p.reshape(B, -1) for p in (p0_conf, p1_conf, p2_conf)], axis=1)
    box_class = jnp.concatenate(
        [p.reshape(B, -1, C) for p in (p0_class, p1_class, p2_class)], axis=1)
    N = bboxes.shape[1]
    pad = (-N) % _LANES
    if pad:
        bboxes = jnp.pad(bboxes, ((0, 0), (0, pad), (0, 0)))
        box_conf = jnp.pad(box_conf, ((0, 0), (0, pad)))
        box_class = jnp.pad(box_class, ((0, 0), (0, pad), (0, 0)))
    Np = N + pad
    rows = Np // _LANES

    class_t = box_class.transpose(0, 2, 1).reshape(B, C, rows, _LANES)
    conf_t = box_conf.reshape(B, rows, _LANES)
    boxes_t = bboxes.transpose(0, 2, 1).reshape(B, 4, rows, _LANES)

    s_all, b_all = _PCALL(
        _nms_stage1_kernel,
        grid=(B, C // _CGRP),
        in_specs=[
            pl.BlockSpec((1, _CGRP, rows, _LANES), lambda b, c: (b, c, 0, 0)),
            pl.BlockSpec((1, rows, _LANES), lambda b, c: (b, 0, 0)),
            pl.BlockSpec((1, 4, rows, _LANES), lambda b, c: (b, 0, 0, 0)),
        ],
        out_specs=[
            pl.BlockSpec((1, 1, _CGRP, _LANES), lambda b, c: (b, c, 0, 0)),
            pl.BlockSpec((1, 1, _CGRP * 4, _LANES), lambda b, c: (b, c, 0, 0)),
        ],
        out_shape=[
            jax.ShapeDtypeStruct((B, C // _CGRP, _CGRP, _LANES), jnp.float32),
            jax.ShapeDtypeStruct(
                (B, C // _CGRP, _CGRP * 4, _LANES), jnp.float32),
        ],
        compiler_params=pltpu.CompilerParams(
            dimension_semantics=("parallel", "parallel")),
    )(class_t, conf_t, boxes_t)
    s_all = s_all.reshape(B, C, _LANES)
    b_all = b_all.reshape(B, C, 4, _LANES)

    scr, cls, box, val = _PCALL(
        _merge_stage2_kernel,
        grid=(B,),
        in_specs=[
            pl.BlockSpec((1, C, _LANES), lambda b: (b, 0, 0)),
            pl.BlockSpec((1, C, 4, _LANES), lambda b: (b, 0, 0, 0)),
        ],
        out_specs=[
            pl.BlockSpec((1, 1, _LANES), lambda b: (b, 0, 0)),
            pl.BlockSpec((1, 1, _LANES), lambda b: (b, 0, 0)),
            pl.BlockSpec((1, 4, _LANES), lambda b: (b, 0, 0)),
            pl.BlockSpec((1, 1, 1), lambda b: (b, 0, 0)),
        ],
        out_shape=[
            jax.ShapeDtypeStruct((B, 1, _LANES), jnp.float32),
            jax.ShapeDtypeStruct((B, 1, _LANES), jnp.float32),
            jax.ShapeDtypeStruct((B, 4, _LANES), jnp.float32),
            jax.ShapeDtypeStruct((B, 1, 1), jnp.int32),
        ],
        compiler_params=pltpu.CompilerParams(
            dimension_semantics=("parallel",)),
    )(s_all, b_all)

    boxes_o = box[:, :, :_MAX_TOTAL].transpose(0, 2, 1)
    scores_o = scr[:, 0, :_MAX_TOTAL]
    classes_o = cls[:, 0, :_MAX_TOTAL]
    valid_o = val[:, 0, 0]
    return boxes_o, scores_o, classes_o, valid_o


# speedup vs baseline: 1.6992x; 1.0440x over previous
import functools

import jax
import jax.numpy as jnp
from jax.experimental import pallas as pl
from jax.experimental.pallas import tpu as pltpu

_NUM_CLASSES = 80
_IOU_THR = 0.5
_SCORE_THR = 0.25
_MAX_PER_CLASS = 100
_MAX_TOTAL = 100
_K_CAND = 200
_LANES = 128

_PCALL = functools.partial(pl.pallas_call)


_CGRP = 16


def _nms_stage1_kernel(class_ref, conf_ref, boxes_ref, s_out_ref, b_out_ref):
    conf = conf_ref[0]
    s0s = tuple(
        jnp.where(class_ref[0, g] * conf > _SCORE_THR,
                  class_ref[0, g] * conf, -1.0)
        for g in range(_CGRP))

    def bis_body(_, st):
        los, his = st
        nlo, nhi = [], []
        for g in range(_CGRP):
            mid = 0.5 * (los[g] + his[g])
            cnt = jnp.sum((s0s[g] >= mid).astype(jnp.int32))
            ge = cnt >= _K_CAND
            nlo.append(jnp.where(ge, mid, los[g]))
            nhi.append(jnp.where(ge, his[g], mid))
        return (tuple(nlo), tuple(nhi))

    lo0 = (jnp.float32(_SCORE_THR),) * _CGRP
    hi0 = (jnp.float32(1.5),) * _CGRP
    los, _ = jax.lax.fori_loop(0, 50, bis_body, (lo0, hi0))
    s_init = tuple(
        jnp.where(s0s[g] >= los[g], s0s[g], -1.0) for g in range(_CGRP))

    y1 = boxes_ref[0, 0]
    x1 = boxes_ref[0, 1]
    y2 = boxes_ref[0, 2]
    x2 = boxes_ref[0, 3]
    area = jnp.maximum(y2 - y1, 0.0) * jnp.maximum(x2 - x1, 0.0)
    lane = jax.lax.broadcasted_iota(jnp.int32, (1, _LANES), 1)

    def body(i, st):
        ss, oss, obs = st
        pick = (lane == i).astype(jnp.float32)
        nss, noss, nobs = [], [], []
        for g in range(_CGRP):
            s = ss[g]
            m = jnp.max(s)
            valid = m > 0.0
            oh = (s == m).astype(jnp.float32)
            sy1 = jnp.sum(oh * y1)
            sx1 = jnp.sum(oh * x1)
            sy2 = jnp.sum(oh * y2)
            sx2 = jnp.sum(oh * x2)
            sarea = jnp.maximum(sy2 - sy1, 0.0) * jnp.maximum(sx2 - sx1, 0.0)
            iy1 = jnp.maximum(y1, sy1)
            ix1 = jnp.maximum(x1, sx1)
            iy2 = jnp.minimum(y2, sy2)
            ix2 = jnp.minimum(x2, sx2)
            inter = jnp.maximum(iy2 - iy1, 0.0) * jnp.maximum(ix2 - ix1, 0.0)
            union = area + sarea - inter
            iou = inter / jnp.maximum(union, 1e-9)
            nss.append(jnp.where((iou > _IOU_THR) | (oh > 0.0), -1.0, s))
            os_, (ob0, ob1, ob2, ob3) = oss[g], obs[g]
            noss.append(os_ + jnp.where(valid, m, 0.0) * pick)
            nobs.append((ob0 + jnp.where(valid, sy1, 0.0) * pick,
                         ob1 + jnp.where(valid, sx1, 0.0) * pick,
                         ob2 + jnp.where(valid, sy2, 0.0) * pick,
                         ob3 + jnp.where(valid, sx2, 0.0) * pick))
        return (tuple(nss), tuple(noss), tuple(nobs))

    z = jnp.zeros((1, _LANES), jnp.float32)
    st0 = (s_init, (z,) * _CGRP, ((z, z, z, z),) * _CGRP)
    _, oss, obs = jax.lax.fori_loop(0, _MAX_PER_CLASS, body, st0)
    s_out_ref[...] = jnp.concatenate(oss, axis=0).reshape(1, 1, _CGRP, _LANES)
    b_out_ref[...] = jnp.concatenate(
        [jnp.concatenate(obs[g], axis=0) for g in range(_CGRP)],
        axis=0).reshape(1, 1, _CGRP * 4, _LANES)


def _merge_stage2_kernel(s_ref, b_ref, scr_ref, cls_ref, box_ref, val_ref):
    S = s_ref[0]
    row = jax.lax.broadcasted_iota(
        jnp.int32, (_NUM_CLASSES, _LANES), 0).astype(jnp.float32)
    lane = jax.lax.broadcasted_iota(jnp.int32, (1, _LANES), 1)
    n_valid = jnp.minimum(jnp.sum((S > 0.0).astype(jnp.int32)), _MAX_TOTAL)

    def body(i, st):
        s, oscr, ocls, ob0, ob1, ob2, ob3 = st
        m = jnp.max(s)
        valid = m > 0.0
        oh = (s == m).astype(jnp.float32)
        c = jnp.sum(oh * row)
        sy1 = jnp.sum(oh * b_ref[0, :, 0, :])
        sx1 = jnp.sum(oh * b_ref[0, :, 1, :])
        sy2 = jnp.sum(oh * b_ref[0, :, 2, :])
        sx2 = jnp.sum(oh * b_ref[0, :, 3, :])
        s = jnp.where(oh > 0.0, -1.0, s)
        pick = (lane == i).astype(jnp.float32)
        oscr = oscr + jnp.where(valid, m, 0.0) * pick
        ocls = ocls + jnp.where(valid, c, 0.0) * pick
        ob0 = ob0 + jnp.where(valid, sy1, 0.0) * pick
        ob1 = ob1 + jnp.where(valid, sx1, 0.0) * pick
        ob2 = ob2 + jnp.where(valid, sy2, 0.0) * pick
        ob3 = ob3 + jnp.where(valid, sx2, 0.0) * pick
        return (s, oscr, ocls, ob0, ob1, ob2, ob3)

    z = jnp.zeros((1, _LANES), jnp.float32)
    _, oscr, ocls, ob0, ob1, ob2, ob3 = jax.lax.fori_loop(
        0, _MAX_TOTAL, body, (S, z, z, z, z, z, z))
    scr_ref[...] = oscr.reshape(1, 1, _LANES)
    cls_ref[...] = ocls.reshape(1, 1, _LANES)
    box_ref[...] = jnp.concatenate(
        [ob0, ob1, ob2, ob3], axis=0).reshape(1, 4, _LANES)
    val_ref[...] = jnp.full((1, 1, 1), n_valid, jnp.int32)


def kernel(p0_boxes, p0_conf, p0_class, p1_boxes, p1_conf, p1_class,
           p2_boxes, p2_conf, p2_class):
    B = p0_boxes.shape[0]
    C = _NUM_CLASSES
    bboxes = jnp.concatenate(
        [p.reshape(B, -1, 4) for p in (p0_boxes, p1_boxes, p2_boxes)], axis=1)
    box_conf = jnp.concatenate(
        [p.reshape(B, -1) for p in (p0_conf, p1_conf, p2_conf)], axis=1)
    box_class = jnp.concatenate(
        [p.reshape(B, -1, C) for p in (p0_class, p1_class, p2_class)], axis=1)
    N = bboxes.shape[1]
    pad = (-N) % _LANES
    if pad:
        bboxes = jnp.pad(bboxes, ((0, 0), (0, pad), (0, 0)))
        box_conf = jnp.pad(box_conf, ((0, 0), (0, pad)))
        box_class = jnp.pad(box_class, ((0, 0), (0, pad), (0, 0)))
    Np = N + pad
    rows = Np // _LANES

    class_t = box_class.transpose(0, 2, 1).reshape(B, C, rows, _LANES)
    conf_t = box_conf.reshape(B, rows, _LANES)
    boxes_t = bboxes.transpose(0, 2, 1).reshape(B, 4, rows, _LANES)

    s_all, b_all = _PCALL(
        _nms_stage1_kernel,
        grid=(B, C // _CGRP),
        in_specs=[
            pl.BlockSpec((1, _CGRP, rows, _LANES), lambda b, c: (b, c, 0, 0)),
            pl.BlockSpec((1, rows, _LANES), lambda b, c: (b, 0, 0)),
            pl.BlockSpec((1, 4, rows, _LANES), lambda b, c: (b, 0, 0, 0)),
        ],
        out_specs=[
            pl.BlockSpec((1, 1, _CGRP, _LANES), lambda b, c: (b, c, 0, 0)),
            pl.BlockSpec((1, 1, _CGRP * 4, _LANES), lambda b, c: (b, c, 0, 0)),
        ],
        out_shape=[
            jax.ShapeDtypeStruct((B, C // _CGRP, _CGRP, _LANES), jnp.float32),
            jax.ShapeDtypeStruct(
                (B, C // _CGRP, _CGRP * 4, _LANES), jnp.float32),
        ],
        compiler_params=pltpu.CompilerParams(
            dimension_semantics=("parallel", "parallel")),
    )(class_t, conf_t, boxes_t)
    s_all = s_all.reshape(B, C, _LANES)
    b_all = b_all.reshape(B, C, 4, _LANES)

    scr, cls, box, val = _PCALL(
        _merge_stage2_kernel,
        grid=(B,),
        in_specs=[
            pl.BlockSpec((1, C, _LANES), lambda b: (b, 0, 0)),
            pl.BlockSpec((1, C, 4, _LANES), lambda b: (b, 0, 0, 0)),
        ],
        out_specs=[
            pl.BlockSpec((1, 1, _LANES), lambda b: (b, 0, 0)),
            pl.BlockSpec((1, 1, _LANES), lambda b: (b, 0, 0)),
            pl.BlockSpec((1, 4, _LANES), lambda b: (b, 0, 0)),
            pl.BlockSpec((1, 1, 1), lambda b: (b, 0, 0)),
        ],
        out_shape=[
            jax.ShapeDtypeStruct((B, 1, _LANES), jnp.float32),
            jax.ShapeDtypeStruct((B, 1, _LANES), jnp.float32),
            jax.ShapeDtypeStruct((B, 4, _LANES), jnp.float32),
            jax.ShapeDtypeStruct((B, 1, 1), jnp.int32),
        ],
        compiler_params=pltpu.CompilerParams(
            dimension_semantics=("parallel",)),
    )(s_all, b_all)

    boxes_o = box[:, :, :_MAX_TOTAL].transpose(0, 2, 1)
    scores_o = scr[:, 0, :_MAX_TOTAL]
    classes_o = cls[:, 0, :_MAX_TOTAL]
    valid_o = val[:, 0, 0]
    return boxes_o, scores_o, classes_o, valid_o


# speedup vs baseline: 1.6998x; 1.0004x over previous
import functools

import jax
import jax.numpy as jnp
from jax.experimental import pallas as pl
from jax.experimental.pallas import tpu as pltpu

_NUM_CLASSES = 80
_IOU_THR = 0.5
_SCORE_THR = 0.25
_MAX_PER_CLASS = 100
_MAX_TOTAL = 100
_K_CAND = 200
_LANES = 128

_PCALL = functools.partial(pl.pallas_call)


_CGRP = 16


def _nms_stage1_kernel(class_ref, conf_ref, boxes_ref, s_out_ref, b_out_ref):
    conf = conf_ref[0]
    s0s = tuple(
        jnp.where(class_ref[0, g] * conf > _SCORE_THR,
                  class_ref[0, g] * conf, -1.0)
        for g in range(_CGRP))

    def bis_body(_, st):
        los, his = st
        nlo, nhi = [], []
        for g in range(_CGRP):
            mid = 0.5 * (los[g] + his[g])
            cnt = jnp.sum((s0s[g] >= mid).astype(jnp.int32))
            ge = cnt >= _K_CAND
            nlo.append(jnp.where(ge, mid, los[g]))
            nhi.append(jnp.where(ge, his[g], mid))
        return (tuple(nlo), tuple(nhi))

    lo0 = (jnp.float32(_SCORE_THR),) * _CGRP
    hi0 = (jnp.float32(1.5),) * _CGRP
    los, _ = jax.lax.fori_loop(0, 50, bis_body, (lo0, hi0))
    s_init = tuple(
        jnp.where(s0s[g] >= los[g], s0s[g], -1.0) for g in range(_CGRP))

    y1 = boxes_ref[0, 0]
    x1 = boxes_ref[0, 1]
    y2 = boxes_ref[0, 2]
    x2 = boxes_ref[0, 3]
    lane = jax.lax.broadcasted_iota(jnp.int32, (1, _LANES), 1)

    def body(i, st):
        ss, oss, obs = st
        pick = (lane == i).astype(jnp.float32)
        nss, noss, nobs = [], [], []
        for g in range(_CGRP):
            s = ss[g]
            m = jnp.max(s)
            valid = m > 0.0
            oh = (s == m).astype(jnp.float32)
            sy1 = jnp.sum(oh * y1)
            sx1 = jnp.sum(oh * x1)
            sy2 = jnp.sum(oh * y2)
            sx2 = jnp.sum(oh * x2)
            sarea = jnp.maximum(sy2 - sy1, 0.0) * jnp.maximum(sx2 - sx1, 0.0)
            iy1 = jnp.maximum(y1, sy1)
            ix1 = jnp.maximum(x1, sx1)
            iy2 = jnp.minimum(y2, sy2)
            ix2 = jnp.minimum(x2, sx2)
            inter = jnp.maximum(iy2 - iy1, 0.0) * jnp.maximum(ix2 - ix1, 0.0)
            area = jnp.maximum(y2 - y1, 0.0) * jnp.maximum(x2 - x1, 0.0)
            union = area + sarea - inter
            iou = inter / jnp.maximum(union, 1e-9)
            nss.append(jnp.where((iou > _IOU_THR) | (oh > 0.0), -1.0, s))
            os_, (ob0, ob1, ob2, ob3) = oss[g], obs[g]
            noss.append(os_ + jnp.where(valid, m, 0.0) * pick)
            nobs.append((ob0 + jnp.where(valid, sy1, 0.0) * pick,
                         ob1 + jnp.where(valid, sx1, 0.0) * pick,
                         ob2 + jnp.where(valid, sy2, 0.0) * pick,
                         ob3 + jnp.where(valid, sx2, 0.0) * pick))
        return (tuple(nss), tuple(noss), tuple(nobs))

    z = jnp.zeros((1, _LANES), jnp.float32)
    st0 = (s_init, (z,) * _CGRP, ((z, z, z, z),) * _CGRP)
    _, oss, obs = jax.lax.fori_loop(0, _MAX_PER_CLASS, body, st0)
    s_out_ref[...] = jnp.concatenate(oss, axis=0).reshape(1, 1, _CGRP, _LANES)
    b_out_ref[...] = jnp.concatenate(
        [jnp.concatenate(obs[g], axis=0) for g in range(_CGRP)],
        axis=0).reshape(1, 1, _CGRP * 4, _LANES)


def _merge_stage2_kernel(s_ref, b_ref, scr_ref, cls_ref, box_ref, val_ref):
    S = s_ref[0]
    row = jax.lax.broadcasted_iota(
        jnp.int32, (_NUM_CLASSES, _LANES), 0).astype(jnp.float32)
    lane = jax.lax.broadcasted_iota(jnp.int32, (1, _LANES), 1)
    n_valid = jnp.minimum(jnp.sum((S > 0.0).astype(jnp.int32)), _MAX_TOTAL)

    def body(i, st):
        s, oscr, ocls, ob0, ob1, ob2, ob3 = st
        m = jnp.max(s)
        valid = m > 0.0
        oh = (s == m).astype(jnp.float32)
        c = jnp.sum(oh * row)
        sy1 = jnp.sum(oh * b_ref[0, :, 0, :])
        sx1 = jnp.sum(oh * b_ref[0, :, 1, :])
        sy2 = jnp.sum(oh * b_ref[0, :, 2, :])
        sx2 = jnp.sum(oh * b_ref[0, :, 3, :])
        s = jnp.where(oh > 0.0, -1.0, s)
        pick = (lane == i).astype(jnp.float32)
        oscr = oscr + jnp.where(valid, m, 0.0) * pick
        ocls = ocls + jnp.where(valid, c, 0.0) * pick
        ob0 = ob0 + jnp.where(valid, sy1, 0.0) * pick
        ob1 = ob1 + jnp.where(valid, sx1, 0.0) * pick
        ob2 = ob2 + jnp.where(valid, sy2, 0.0) * pick
        ob3 = ob3 + jnp.where(valid, sx2, 0.0) * pick
        return (s, oscr, ocls, ob0, ob1, ob2, ob3)

    z = jnp.zeros((1, _LANES), jnp.float32)
    _, oscr, ocls, ob0, ob1, ob2, ob3 = jax.lax.fori_loop(
        0, _MAX_TOTAL, body, (S, z, z, z, z, z, z))
    scr_ref[...] = oscr.reshape(1, 1, _LANES)
    cls_ref[...] = ocls.reshape(1, 1, _LANES)
    box_ref[...] = jnp.concatenate(
        [ob0, ob1, ob2, ob3], axis=0).reshape(1, 4, _LANES)
    val_ref[...] = jnp.full((1, 1, 1), n_valid, jnp.int32)


def kernel(p0_boxes, p0_conf, p0_class, p1_boxes, p1_conf, p1_class,
           p2_boxes, p2_conf, p2_class):
    B = p0_boxes.shape[0]
    C = _NUM_CLASSES
    bboxes = jnp.concatenate(
        [p.reshape(B, -1, 4) for p in (p0_boxes, p1_boxes, p2_boxes)], axis=1)
    box_conf = jnp.concatenate(
        [p.reshape(B, -1) for p in (p0_conf, p1_conf, p2_conf)], axis=1)
    box_class = jnp.concatenate(
        [p.reshape(B, -1, C) for p in (p0_class, p1_class, p2_class)], axis=1)
    N = bboxes.shape[1]
    pad = (-N) % _LANES
    if pad:
        bboxes = jnp.pad(bboxes, ((0, 0), (0, pad), (0, 0)))
        box_conf = jnp.pad(box_conf, ((0, 0), (0, pad)))
        box_class = jnp.pad(box_class, ((0, 0), (0, pad), (0, 0)))
    Np = N + pad
    rows = Np // _LANES

    class_t = box_class.transpose(0, 2, 1).reshape(B, C, rows, _LANES)
    conf_t = box_conf.reshape(B, rows, _LANES)
    boxes_t = bboxes.transpose(0, 2, 1).reshape(B, 4, rows, _LANES)

    s_all, b_all = _PCALL(
        _nms_stage1_kernel,
        grid=(B, C // _CGRP),
        in_specs=[
            pl.BlockSpec((1, _CGRP, rows, _LANES), lambda b, c: (b, c, 0, 0)),
            pl.BlockSpec((1, rows, _LANES), lambda b, c: (b, 0, 0)),
            pl.BlockSpec((1, 4, rows, _LANES), lambda b, c: (b, 0, 0, 0)),
        ],
        out_specs=[
            pl.BlockSpec((1, 1, _CGRP, _LANES), lambda b, c: (b, c, 0, 0)),
            pl.BlockSpec((1, 1, _CGRP * 4, _LANES), lambda b, c: (b, c, 0, 0)),
        ],
        out_shape=[
            jax.ShapeDtypeStruct((B, C // _CGRP, _CGRP, _LANES), jnp.float32),
            jax.ShapeDtypeStruct(
                (B, C // _CGRP, _CGRP * 4, _LANES), jnp.float32),
        ],
        compiler_params=pltpu.CompilerParams(
            dimension_semantics=("parallel", "parallel")),
    )(class_t, conf_t, boxes_t)
    s_all = s_all.reshape(B, C, _LANES)
    b_all = b_all.reshape(B, C, 4, _LANES)

    scr, cls, box, val = _PCALL(
        _merge_stage2_kernel,
        grid=(B,),
        in_specs=[
            pl.BlockSpec((1, C, _LANES), lambda b: (b, 0, 0)),
            pl.BlockSpec((1, C, 4, _LANES), lambda b: (b, 0, 0, 0)),
        ],
        out_specs=[
            pl.BlockSpec((1, 1, _LANES), lambda b: (b, 0, 0)),
            pl.BlockSpec((1, 1, _LANES), lambda b: (b, 0, 0)),
            pl.BlockSpec((1, 4, _LANES), lambda b: (b, 0, 0)),
            pl.BlockSpec((1, 1, 1), lambda b: (b, 0, 0)),
        ],
        out_shape=[
            jax.ShapeDtypeStruct((B, 1, _LANES), jnp.float32),
            jax.ShapeDtypeStruct((B, 1, _LANES), jnp.float32),
            jax.ShapeDtypeStruct((B, 4, _LANES), jnp.float32),
            jax.ShapeDtypeStruct((B, 1, 1), jnp.int32),
        ],
        compiler_params=pltpu.CompilerParams(
            dimension_semantics=("parallel",)),
    )(s_all, b_all)

    boxes_o = box[:, :, :_MAX_TOTAL].transpose(0, 2, 1)
    scores_o = scr[:, 0, :_MAX_TOTAL]
    classes_o = cls[:, 0, :_MAX_TOTAL]
    valid_o = val[:, 0, 0]
    return boxes_o, scores_o, classes_o, valid_o
